# Initial kernel scaffold; baseline (speedup 1.0000x reference)
import jax
import jax.numpy as jnp
from jax.experimental import pallas as pl


def kernel(from_square_idx, to_square_idx, action_kind_idx, promotion_idx, actor_piece_idx, actor_class_idx, target_piece_idx, target_class_idx, tag_idx, value_namespace_idx, value_label_idx, value_path_idx, value_depth_idx, value_position_idx, action_features, value_features, square_table, kind_table, promo_table, actor_piece_table, actor_class_table, target_piece_table, target_class_table, tag_table, ns_table, label_table, path_table, depth_table, pos_table, W1f, b1f, W2f, b2f, W1v, b1v, W2v, b2v, gamma, beta, Wo1, bo1, Wo2, bo2):
    raise NotImplementedError("write your pallas kernel here")


# speedup vs baseline: 4.4610x; 4.4610x over previous
import functools

import jax
import jax.numpy as jnp
from jax.experimental import pallas as pl
from jax.experimental.pallas import tpu as pltpu

N = 4096
P = 8
T = 4
D = 256
INNER = 512

_SIZES = [65, 64, 8, 129, 65, 129, 65, 65, 129, 257, 257, 17, 65]
_OFFS = []
_acc = 0
for _s in _SIZES:
    _OFFS.append(_acc)
    _acc += _s
VTOT_RAW = _acc
VTOT = 1344
_PAD_IDX = VTOT_RAW

(OFF_SQ, OFF_KIND, OFF_PROMO, OFF_AP, OFF_AC, OFF_TP, OFF_TC, OFF_TAG,
 OFF_NS, OFF_LABEL, OFF_PATH, OFF_DEPTH, OFF_POS) = _OFFS

BN = 256
NUM_COLS = 64

_COL_W = [1.0] * 12 + [1.0 / P] * (5 * P) + [0.0] * (NUM_COLS - 52)


def _build_gidx(from_square_idx, to_square_idx, action_kind_idx, promotion_idx,
                actor_piece_idx, actor_class_idx, target_piece_idx,
                target_class_idx, tag_idx, value_namespace_idx, value_label_idx,
                value_path_idx, value_depth_idx, value_position_idx):
    cols = [
        from_square_idx + OFF_SQ,
        to_square_idx + OFF_SQ,
        action_kind_idx + OFF_KIND,
        promotion_idx + OFF_PROMO,
        actor_piece_idx + OFF_AP,
        actor_class_idx + OFF_AC,
        target_piece_idx + OFF_TP,
        target_class_idx + OFF_TC,
    ]
    cols = [c[:, None] for c in cols]
    cols.append(tag_idx + OFF_TAG)
    cols.append(value_namespace_idx + OFF_NS)
    cols.append(value_label_idx + OFF_LABEL)
    cols.append(value_path_idx + OFF_PATH)
    cols.append(value_depth_idx + OFF_DEPTH)
    cols.append(value_position_idx + OFF_POS)
    g = jnp.concatenate(cols, axis=1).astype(jnp.int32)
    pad = jnp.full((g.shape[0], NUM_COLS - g.shape[1]), _PAD_IDX, jnp.int32)
    return jnp.concatenate([g, pad], axis=1)


def _tc_body(g_ref, af_ref, vf_ref, tcat_ref,
             w1f_ref, b1f_ref, w2f_ref, b2f_ref,
             w1v_ref, b1v_ref, w2v_ref, b2v_ref,
             gamma_ref, beta_ref, wo1_ref, bo1_ref, wo2_ref, bo2_ref,
             out_ref):
    g = g_ref[...]
    iota = jax.lax.broadcasted_iota(jnp.int32, (BN, VTOT), 1)
    c = jnp.zeros((BN, VTOT), jnp.float32)
    for j in range(52):
        c = c + jnp.where(iota == g[:, j][:, None],
                          jnp.float32(_COL_W[j]), jnp.float32(0.0))
    emb = jnp.dot(c, tcat_ref[...], preferred_element_type=jnp.float32)

    hf = jax.nn.gelu(jnp.dot(af_ref[...], w1f_ref[...],
                             preferred_element_type=jnp.float32) + b1f_ref[...])
    flag = jnp.dot(hf, w2f_ref[...],
                   preferred_element_type=jnp.float32) + b2f_ref[...]

    vf = vf_ref[...]
    acc = jnp.zeros((BN, INNER), jnp.float32)
    for p in range(P):
        xp = vf[:, p * 10:(p + 1) * 10]
        acc = acc + jax.nn.gelu(
            jnp.dot(xp, w1v_ref[...], preferred_element_type=jnp.float32)
            + b1v_ref[...])
    val = jnp.dot(acc * jnp.float32(1.0 / P), w2v_ref[...],
                  preferred_element_type=jnp.float32) + b2v_ref[...]

    h = emb + flag + val
    mu = jnp.mean(h, axis=-1, keepdims=True)
    dev = h - mu
    var = jnp.mean(dev * dev, axis=-1, keepdims=True)
    hn = dev * jax.lax.rsqrt(var + jnp.float32(1e-5)) * gamma_ref[...] \
        + beta_ref[...]

    ho = jax.nn.gelu(jnp.dot(hn, wo1_ref[...],
                             preferred_element_type=jnp.float32) + bo1_ref[...])
    out_ref[...] = jnp.dot(ho, wo2_ref[...],
                           preferred_element_type=jnp.float32) + bo2_ref[...]


def kernel(from_square_idx, to_square_idx, action_kind_idx, promotion_idx,
           actor_piece_idx, actor_class_idx, target_piece_idx,
           target_class_idx, tag_idx, value_namespace_idx, value_label_idx,
           value_path_idx, value_depth_idx, value_position_idx,
           action_features, value_features,
           square_table, kind_table, promo_table, actor_piece_table,
           actor_class_table, target_piece_table, target_class_table,
           tag_table, ns_table, label_table, path_table, depth_table,
           pos_table,
           W1f, b1f, W2f, b2f, W1v, b1v, W2v, b2v,
           gamma, beta, Wo1, bo1, Wo2, bo2):
    gidx = _build_gidx(from_square_idx, to_square_idx, action_kind_idx,
                       promotion_idx, actor_piece_idx, actor_class_idx,
                       target_piece_idx, target_class_idx, tag_idx,
                       value_namespace_idx, value_label_idx, value_path_idx,
                       value_depth_idx, value_position_idx)
    tcat = jnp.concatenate([
        square_table, kind_table, promo_table, actor_piece_table,
        actor_class_table, target_piece_table, target_class_table, tag_table,
        ns_table, label_table, path_table, depth_table, pos_table,
        jnp.zeros((VTOT - VTOT_RAW, D), jnp.float32)], axis=0)
    vf = value_features.reshape(N, P * 10)

    grid = (N // BN,)
    row_spec = lambda w: pl.BlockSpec((BN, w), lambda i: (i, 0))
    full = lambda a: pl.BlockSpec(a.shape, lambda i: (0,) * a.ndim)

    out = pl.pallas_call(
        _tc_body,
        grid=grid,
        in_specs=[
            row_spec(NUM_COLS),
            row_spec(8),
            row_spec(P * 10),
            full(tcat), full(W1f), full(b1f), full(W2f), full(b2f),
            full(W1v), full(b1v), full(W2v), full(b2v),
            full(gamma), full(beta), full(Wo1), full(bo1), full(Wo2),
            full(bo2),
        ],
        out_specs=pl.BlockSpec((BN, D), lambda i: (i, 0)),
        out_shape=jax.ShapeDtypeStruct((N, D), jnp.float32),
        compiler_params=pltpu.CompilerParams(
            dimension_semantics=("arbitrary",)),
    )(gidx, action_features, vf, tcat, W1f, b1f, W2f, b2f,
      W1v, b1v, W2v, b2v, gamma, beta, Wo1, bo1, Wo2, bo2)
    return out
